# baseline (speedup 1.0000x reference)
import functools

import jax
import jax.numpy as jnp
from jax import lax
from jax.experimental import pallas as pl
from jax.experimental.pallas import tpu as pltpu
from jax.experimental.pallas import tpu_sc as plsc

N = 10000
E = 320000
HD = 256
G = 64


_ROWS = 1000
_EROWS = 1000


def _pre_body(x_ref, wa_ref, wb_ref, pa_ref, pb_ref):
    xb = x_ref[...]
    pa_ref[...] = jnp.dot(xb, wa_ref[...], preferred_element_type=jnp.float32)
    pb_ref[...] = jnp.dot(xb, wb_ref[...], preferred_element_type=jnp.float32)


def _pre_tc(xin, wa, wb):
    k = xin.shape[1]
    return pl.pallas_call(
        _pre_body,
        grid=(N // _ROWS,),
        in_specs=[
            pl.BlockSpec((_ROWS, k), lambda i: (i, 0)),
            pl.BlockSpec((k, HD), lambda i: (0, 0)),
            pl.BlockSpec((k, HD), lambda i: (0, 0)),
        ],
        out_specs=[
            pl.BlockSpec((_ROWS, HD), lambda i: (i, 0)),
            pl.BlockSpec((_ROWS, HD), lambda i: (i, 0)),
        ],
        out_shape=[
            jax.ShapeDtypeStruct((N, HD), jnp.float32),
            jax.ShapeDtypeStruct((N, HD), jnp.float32),
        ],
    )(xin, wa, wb)


def _edge_body(ga_ref, gb_ref, ea_ref, wc_ref, b1_ref, w2_ref, b2_ref, out_ref):
    z = (
        ga_ref[...]
        + gb_ref[...]
        + jnp.dot(ea_ref[...], wc_ref[...], preferred_element_type=jnp.float32)
        + b1_ref[...]
    )
    z = jnp.maximum(z, 0.0)
    out_ref[...] = jnp.dot(z, w2_ref[...], preferred_element_type=jnp.float32) + b2_ref[...]


def _edge_tc(ga, gb, ea, wc, b1, w2, b2):
    ed = ea.shape[1]
    return pl.pallas_call(
        _edge_body,
        grid=(E // _EROWS,),
        in_specs=[
            pl.BlockSpec((_EROWS, HD), lambda i: (i, 0)),
            pl.BlockSpec((_EROWS, HD), lambda i: (i, 0)),
            pl.BlockSpec((_EROWS, ed), lambda i: (i, 0)),
            pl.BlockSpec((ed, HD), lambda i: (0, 0)),
            pl.BlockSpec((1, HD), lambda i: (0, 0)),
            pl.BlockSpec((HD, HD), lambda i: (0, 0)),
            pl.BlockSpec((1, HD), lambda i: (0, 0)),
        ],
        out_specs=pl.BlockSpec((_EROWS, HD), lambda i: (i, 0)),
        out_shape=jax.ShapeDtypeStruct((E, HD), jnp.float32),
    )(ga, gb, ea, wc, b1, w2, b2)


def _node_body(x_ref, ag_ref, w1a_ref, w1b_ref, b1_ref, w2_ref, b2_ref,
               out_ref):
    t = (
        jnp.dot(x_ref[...], w1a_ref[...], preferred_element_type=jnp.float32)
        + jnp.dot(ag_ref[...], w1b_ref[...], preferred_element_type=jnp.float32)
        + b1_ref[...]
    )
    t = jnp.maximum(t, 0.0)
    out_ref[...] = jnp.maximum(
        jnp.dot(t, w2_ref[...], preferred_element_type=jnp.float32) + b2_ref[...], 0.0
    )


def _node_tc(xin, aggr, w1a, w1b, b1, w2, b2):
    k = xin.shape[1]
    return pl.pallas_call(
        _node_body,
        grid=(N // _ROWS,),
        in_specs=[
            pl.BlockSpec((_ROWS, k), lambda i: (i, 0)),
            pl.BlockSpec((_ROWS, HD), lambda i: (i, 0)),
            pl.BlockSpec((k, HD), lambda i: (0, 0)),
            pl.BlockSpec((HD, HD), lambda i: (0, 0)),
            pl.BlockSpec((1, HD), lambda i: (0, 0)),
            pl.BlockSpec((HD, HD), lambda i: (0, 0)),
            pl.BlockSpec((1, HD), lambda i: (0, 0)),
        ],
        out_specs=pl.BlockSpec((_ROWS, HD), lambda i: (i, 0)),
        out_shape=jax.ShapeDtypeStruct((N, HD), jnp.float32),
    )(xin, aggr, w1a, w1b, b1, w2, b2)


def _pool_body(h_ref, batch_ref, g_ref, w1a_ref, w1b_ref, b1_ref, w2_ref, b2_ref,
               out_ref, sums_ref, counts_ref):
    i = pl.program_id(0)

    @pl.when(i == 0)
    def _init():
        sums_ref[...] = jnp.zeros_like(sums_ref)
        counts_ref[...] = jnp.zeros_like(counts_ref)

    b = batch_ref[...].reshape(1, _ROWS)
    seg = lax.broadcasted_iota(jnp.int32, (G, _ROWS), 0)
    onehot = (seg == b).astype(jnp.float32)
    sums_ref[...] += jnp.dot(onehot, h_ref[...], preferred_element_type=jnp.float32)
    cnt = jnp.sum(onehot, axis=1, keepdims=True)
    counts_ref[...] += jnp.broadcast_to(cnt, (G, 128))

    @pl.when(i == (N // _ROWS) - 1)
    def _final():
        c = counts_ref[...][:, 0:1]
        pooled = sums_ref[...] / jnp.maximum(c, 1.0)
        t = (
            jnp.dot(pooled, w1a_ref[...], preferred_element_type=jnp.float32)
            + jnp.dot(g_ref[...], w1b_ref[...], preferred_element_type=jnp.float32)
            + b1_ref[...]
        )
        t = jnp.maximum(t, 0.0)
        out_ref[...] = (
            jnp.dot(t, w2_ref[...], preferred_element_type=jnp.float32) + b2_ref[...]
        )


def _pool_tc(h, batch3d, gfeat, w1a, w1b, b1, w2, b2):
    gd = gfeat.shape[1]
    nc = w2.shape[1]
    return pl.pallas_call(
        _pool_body,
        grid=(N // _ROWS,),
        in_specs=[
            pl.BlockSpec((_ROWS, HD), lambda i: (i, 0)),
            pl.BlockSpec((1, 1, _ROWS), lambda i: (i, 0, 0)),
            pl.BlockSpec((G, gd), lambda i: (0, 0)),
            pl.BlockSpec((HD, HD), lambda i: (0, 0)),
            pl.BlockSpec((gd, HD), lambda i: (0, 0)),
            pl.BlockSpec((1, HD), lambda i: (0, 0)),
            pl.BlockSpec((HD, nc), lambda i: (0, 0)),
            pl.BlockSpec((1, nc), lambda i: (0, 0)),
        ],
        out_specs=pl.BlockSpec((G, nc), lambda i: (0, 0)),
        out_shape=jax.ShapeDtypeStruct((G, nc), jnp.float32),
        scratch_shapes=[
            pltpu.VMEM((G, HD), jnp.float32),
            pltpu.VMEM((G, 128), jnp.float32),
        ],
    )(h, batch3d, gfeat, w1a, w1b, b1, w2, b2)


_SC_MESH = plsc.VectorSubcoreMesh(core_axis_name="c", subcore_axis_name="s")
_NW = 32
_CH = 80

_G_PER_W = E // _NW
_G_CHUNKS = _G_PER_W // _CH

_S_PER_T = (E // 2) // 16
_S_CHUNKS = _S_PER_T // _CH
_ZROWS = 1000


@functools.partial(
    pl.kernel,
    mesh=_SC_MESH,
    out_type=[
        jax.ShapeDtypeStruct((E, HD), jnp.float32),
        jax.ShapeDtypeStruct((E, HD), jnp.float32),
    ],
    scratch_types=[
        pltpu.VMEM((_CH,), jnp.int32),
        pltpu.VMEM((_CH,), jnp.int32),
        pltpu.VMEM((_CH,), jnp.int32),
        pltpu.VMEM((_CH,), jnp.int32),
        pltpu.VMEM((_CH, HD), jnp.float32),
        pltpu.VMEM((_CH, HD), jnp.float32),
        pltpu.VMEM((_CH, HD), jnp.float32),
        pltpu.VMEM((_CH, HD), jnp.float32),
        pltpu.SemaphoreType.DMA((2,)),
        pltpu.SemaphoreType.DMA((2,)),
        pltpu.SemaphoreType.DMA((2,)),
        pltpu.SemaphoreType.DMA((2,)),
    ],
)
def _sc_gather(pa_hbm, pb_hbm, dst_hbm, src_hbm, ga_hbm, gb_hbm,
               idxd0, idxd1, idxs0, idxs1, ra0, ra1, rb0, rb1,
               sga, sgb, swa, swb):
    wid = lax.axis_index("s") * 2 + lax.axis_index("c")
    start = wid * _G_PER_W
    idxd = (idxd0, idxd1)
    idxs = (idxs0, idxs1)
    ra = (ra0, ra1)
    rb = (rb0, rb1)

    def load_idx_and_gather(c, p):
        base = start + c * _CH
        pltpu.sync_copy(dst_hbm.at[pl.ds(base, _CH)], idxd[p])
        pltpu.sync_copy(src_hbm.at[pl.ds(base, _CH)], idxs[p])
        pltpu.async_copy(pa_hbm.at[idxd[p]], ra[p], sga.at[p])
        pltpu.async_copy(pb_hbm.at[idxs[p]], rb[p], sgb.at[p])

    def wait_gather(p):
        pltpu.make_async_copy(ga_hbm.at[pl.ds(0, _CH)], ra[p], sga.at[p]).wait()
        pltpu.make_async_copy(gb_hbm.at[pl.ds(0, _CH)], rb[p], sgb.at[p]).wait()

    def issue_wb(c, p):
        base = start + c * _CH
        pltpu.async_copy(ra[p], ga_hbm.at[pl.ds(base, _CH)], swa.at[p])
        pltpu.async_copy(rb[p], gb_hbm.at[pl.ds(base, _CH)], swb.at[p])

    def wait_wb(p):
        pltpu.make_async_copy(ra[p], ga_hbm.at[pl.ds(0, _CH)], swa.at[p]).wait()
        pltpu.make_async_copy(rb[p], gb_hbm.at[pl.ds(0, _CH)], swb.at[p]).wait()

    load_idx_and_gather(0, 0)
    load_idx_and_gather(1, 1)

    def body(g, carry):
        c0 = 2 * g
        wait_gather(0)
        issue_wb(c0, 0)
        wait_gather(1)
        issue_wb(c0 + 1, 1)
        wait_wb(0)
        load_idx_and_gather(c0 + 2, 0)
        wait_wb(1)
        load_idx_and_gather(c0 + 3, 1)
        return carry

    lax.fori_loop(0, (_G_CHUNKS - 3) // 2, body, 0)

    wait_gather(0)
    issue_wb(_G_CHUNKS - 3, 0)
    wait_gather(1)
    issue_wb(_G_CHUNKS - 2, 1)
    wait_wb(0)
    load_idx_and_gather(_G_CHUNKS - 1, 0)
    wait_gather(0)
    issue_wb(_G_CHUNKS - 1, 0)
    wait_wb(0)
    wait_wb(1)


_OWN = 320
_NOUT = 32 * _OWN
_SCCH = 512
_SC_CHUNKS = E // _SCCH
_LROWS = _SCCH // 128
_REG = E + 128
_DUMP = E + 64
_FB = 80

_DN = jax.lax.GatherDimensionNumbers(
    offset_dims=(), collapsed_slice_dims=(0,), start_index_map=(0,)
)


_PB = 256
_PBLK = E // _PB


def _pos_body(dst_ref, pos_ref, cnt_ref, carry_ref):
    i = pl.program_id(0)

    @pl.when(i == 0)
    def _init():
        carry_ref[...] = jnp.zeros_like(carry_ref)

    d = dst_ref[...].reshape(1, _PB)
    owner = d // _OWN
    io32 = lax.broadcasted_iota(jnp.int32, (32, _PB), 0)
    onehot = (io32 == owner).astype(jnp.float32)
    r_io = lax.broadcasted_iota(jnp.int32, (_PB, _PB), 0)
    c_io = lax.broadcasted_iota(jnp.int32, (_PB, _PB), 1)
    lt = (r_io <= c_io).astype(jnp.float32)
    prefix = jnp.dot(onehot, lt, preferred_element_type=jnp.float32)
    carry = carry_ref[...][:, 0:1]
    pos_f = jnp.sum(onehot * (prefix + carry), axis=0, keepdims=True) - 1.0
    posg = owner * _REG + pos_f.astype(jnp.int32)
    pos_ref[...] = posg.reshape(1, 1, _PB)
    new_carry = carry + prefix[:, _PB - 1 : _PB]
    carry_ref[...] = jnp.broadcast_to(new_carry, (32, 128))

    @pl.when(i == _PBLK - 1)
    def _final():
        cnt_ref[...] = jnp.broadcast_to(new_carry, (32, 128))


def _pos_tc(dst3d):
    return pl.pallas_call(
        _pos_body,
        grid=(_PBLK,),
        in_specs=[pl.BlockSpec((1, 1, _PB), lambda i: (i, 0, 0))],
        out_specs=[
            pl.BlockSpec((1, 1, _PB), lambda i: (i, 0, 0)),
            pl.BlockSpec((32, 128), lambda i: (0, 0)),
        ],
        out_shape=[
            jax.ShapeDtypeStruct((_PBLK, 1, _PB), jnp.int32),
            jax.ShapeDtypeStruct((32, 128), jnp.float32),
        ],
        scratch_shapes=[pltpu.VMEM((32, 128), jnp.float32)],
    )(dst3d)


def _accumulate_list(m_hbm, dst_hbm, list_hbm, accf, idxb, dvb, rowf, sg,
                     region0, node0, cnt):
    nfull = lax.div(cnt, _FB)
    rem = cnt - nfull * _FB
    cntv16 = jnp.full((16,), cnt, jnp.int32)
    iota = jax.lax.broadcasted_iota(jnp.int32, (16,), 0)

    def add_row(g, l, r):
        base = r * HD
        for q in range(HD // 16):
            plsc.addupdate(
                accf.at[pl.ds(base + q * 16, 16)],
                rowf[g * 16 + l, pl.ds(q * 16, 16)],
            )

    def batch(ch, carry):
        pltpu.sync_copy(list_hbm.at[pl.ds(region0 + ch * _FB, _FB)], idxb)
        cpr = pltpu.async_copy(m_hbm.at[idxb], rowf, sg)
        pltpu.sync_copy(dst_hbm.at[idxb], dvb)
        cpr.wait()

        def acc_group(g, c2):
            dv = dvb[pl.ds(g * 16, 16)]
            for l in range(16):
                add_row(g, l, dv[l] - node0)
            return c2

        lax.fori_loop(0, _FB // 16, acc_group, 0)
        return carry

    lax.fori_loop(0, nfull, batch, 0)

    @pl.when(rem > 0)
    def _tail():
        pltpu.sync_copy(list_hbm.at[pl.ds(region0 + nfull * _FB, _FB)], idxb)
        for gq in range(_FB // 16):
            posv = nfull * _FB + gq * 16 + iota
            v = idxb[pl.ds(gq * 16, 16)]
            idxb[pl.ds(gq * 16, 16)] = jnp.where(posv < cntv16, v, 0)
        cpr = pltpu.async_copy(m_hbm.at[idxb], rowf, sg)
        pltpu.sync_copy(dst_hbm.at[idxb], dvb)
        cpr.wait()

        def acc_group(g, c2):
            dv = dvb[pl.ds(g * 16, 16)]
            for l in range(16):
                r = dv[l] - node0

                @pl.when(g * 16 + l < rem)
                def _(r=r, g=g, l=l):
                    add_row(g, l, r)
            return c2

        lax.fori_loop(0, _FB // 16, acc_group, 0)


def _zero_acc(accf):
    zero16f = jnp.zeros((16,), jnp.float32)

    def zacc(i, carry):
        accf[pl.ds(i * 16, 16)] = zero16f
        return carry

    lax.fori_loop(0, _OWN * HD // 16, zacc, 0)


_LW_CH = 80
_LW_PER_W = E // 32
_LW_CHUNKS = _LW_PER_W // _LW_CH


@functools.partial(
    pl.kernel,
    mesh=_SC_MESH,
    out_type=jax.ShapeDtypeStruct((32 * _REG,), jnp.int32),
    scratch_types=[
        pltpu.VMEM((_LW_CH,), jnp.int32),
        pltpu.VMEM((_LW_CH,), jnp.int32),
        pltpu.VMEM((_LW_CH,), jnp.int32),
        pltpu.VMEM((_LW_CH,), jnp.int32),
        pltpu.SemaphoreType.DMA((2,)),
    ],
)
def _sc_listwrite(pos_hbm, list_hbm, pb0, pb1, vb0, vb1, ssem):
    w = lax.axis_index("s") * 2 + lax.axis_index("c")
    start = w * _LW_PER_W
    pb = (pb0, pb1)
    vb = (vb0, vb1)
    iota = jax.lax.broadcasted_iota(jnp.int32, (16,), 0)

    def load_and_scatter(c, p):
        base = start + c * _LW_CH
        pltpu.sync_copy(pos_hbm.at[pl.ds(base, _LW_CH)], pb[p])
        for j in range(_LW_CH // 16):
            vb[p][pl.ds(j * 16, 16)] = base + j * 16 + iota
        pltpu.async_copy(vb[p], list_hbm.at[pb[p]], ssem.at[p])

    def wait_sc(p):
        pltpu.make_async_copy(vb[p], list_hbm.at[pl.ds(0, _LW_CH)],
                              ssem.at[p]).wait()

    load_and_scatter(0, 0)
    load_and_scatter(1, 1)

    def body(g, carry):
        c0 = 2 * g
        wait_sc(0)
        load_and_scatter(c0 + 2, 0)
        wait_sc(1)
        load_and_scatter(c0 + 3, 1)
        return carry

    lax.fori_loop(0, (_LW_CHUNKS - 3) // 2, body, 0)

    wait_sc(0)
    load_and_scatter(_LW_CHUNKS - 1, 0)
    wait_sc(1)
    wait_sc(0)


@functools.partial(
    pl.kernel,
    mesh=_SC_MESH,
    out_type=jax.ShapeDtypeStruct((_NOUT * HD,), jnp.float32),
    scratch_types=[
        pltpu.VMEM((_OWN * HD,), jnp.float32),
        pltpu.VMEM((16,), jnp.int32),
        pltpu.VMEM((_FB,), jnp.int32),
        pltpu.VMEM((_FB,), jnp.int32),
        pltpu.VMEM((_FB, HD), jnp.float32),
        pltpu.SemaphoreType.DMA,
    ],
)
def _sc_scatter_replay(m_hbm, dst_hbm, list_hbm, cnt_hbm, aggr_hbm,
                       accf, cbuf, idxb, dvb, rowf, sg):
    w = lax.axis_index("s") * 2 + lax.axis_index("c")
    node0 = w * _OWN
    region0 = w * _REG

    _zero_acc(accf)
    pltpu.sync_copy(cnt_hbm.at[pl.ds(w * 16, 16)], cbuf)
    cnt = cbuf[pl.ds(0, 16)][0]
    _accumulate_list(m_hbm, dst_hbm, list_hbm, accf, idxb, dvb, rowf, sg,
                     region0, node0, cnt)
    pltpu.sync_copy(accf, aggr_hbm.at[pl.ds(node0 * HD, _OWN * HD)])


def _layer(xin, src, dst, edge_attr, lists, eW1, eb1, eW2, eb2, nW1, nb1, nW2, nb2):
    k = xin.shape[1]
    pa, pb = _pre_tc(xin, eW1[:k], eW1[k:2 * k])
    ga, gb = _sc_gather(pa, pb, dst, src)
    m = _edge_tc(ga, gb, edge_attr, eW1[2 * k:], eb1.reshape(1, HD), eW2,
                 eb2.reshape(1, HD))
    aggr_flat = _sc_scatter_replay(m, dst, lists[0], lists[1])
    aggr = aggr_flat.reshape(_NOUT, HD)[:N]
    return _node_tc(xin, aggr, nW1[:k], nW1[k:], nb1.reshape(1, HD), nW2,
                    nb2.reshape(1, HD))


def kernel(x, edge_index, edge_attr, batch, global_features,
           l1_eW1, l1_eb1, l1_eW2, l1_eb2, l1_nW1, l1_nb1, l1_nW2, l1_nb2,
           l2_eW1, l2_eb1, l2_eW2, l2_eb2, l2_nW1, l2_nb1, l2_nW2, l2_nb2,
           cW1, cb1, cW2, cb2):
    src = edge_index[0]
    dst = edge_index[1]

    pos3, cnts_f = _pos_tc(dst.reshape(_PBLK, 1, _PB))
    elist = _sc_listwrite(pos3.reshape(E))
    cnts = cnts_f[:, :16].astype(jnp.int32).reshape(32 * 16)
    lists = (elist, cnts)

    h = _layer(x, src, dst, edge_attr, lists,
               l1_eW1, l1_eb1, l1_eW2, l1_eb2, l1_nW1, l1_nb1,
               l1_nW2, l1_nb2)
    h = _layer(h, src, dst, edge_attr, lists,
               l2_eW1, l2_eb1, l2_eW2, l2_eb2, l2_nW1, l2_nb1,
               l2_nW2, l2_nb2)

    batch3d = batch.reshape(N // _ROWS, 1, _ROWS)
    nc = cW2.shape[1]
    return _pool_tc(h, batch3d, global_features, cW1[:HD], cW1[HD:],
                    cb1.reshape(1, HD), cW2, cb2.reshape(1, nc))


# speedup vs baseline: 1.0763x; 1.0763x over previous
import functools

import jax
import jax.numpy as jnp
from jax import lax
from jax.experimental import pallas as pl
from jax.experimental.pallas import tpu as pltpu
from jax.experimental.pallas import tpu_sc as plsc

N = 10000
E = 320000
HD = 256
G = 64


_ROWS = 1000
_EROWS = 1000


def _pre_body(x_ref, wa_ref, wb_ref, pa_ref, pb_ref):
    xb = x_ref[...]
    pa_ref[...] = jnp.dot(xb, wa_ref[...], preferred_element_type=jnp.float32)
    pb_ref[...] = jnp.dot(xb, wb_ref[...], preferred_element_type=jnp.float32)


def _pre_tc(xin, wa, wb):
    k = xin.shape[1]
    return pl.pallas_call(
        _pre_body,
        grid=(N // _ROWS,),
        in_specs=[
            pl.BlockSpec((_ROWS, k), lambda i: (i, 0)),
            pl.BlockSpec((k, HD), lambda i: (0, 0)),
            pl.BlockSpec((k, HD), lambda i: (0, 0)),
        ],
        out_specs=[
            pl.BlockSpec((_ROWS, HD), lambda i: (i, 0)),
            pl.BlockSpec((_ROWS, HD), lambda i: (i, 0)),
        ],
        out_shape=[
            jax.ShapeDtypeStruct((N, HD), jnp.float32),
            jax.ShapeDtypeStruct((N, HD), jnp.float32),
        ],
    )(xin, wa, wb)


def _edge_body(ga_ref, gb_ref, ea_ref, wc_ref, b1_ref, w2_ref, b2_ref, out_ref):
    z = (
        ga_ref[...]
        + gb_ref[...]
        + jnp.dot(ea_ref[...], wc_ref[...], preferred_element_type=jnp.float32)
        + b1_ref[...]
    )
    z = jnp.maximum(z, 0.0)
    out_ref[...] = jnp.dot(z, w2_ref[...], preferred_element_type=jnp.float32) + b2_ref[...]


def _edge_tc(ga, gb, ea, wc, b1, w2, b2):
    ed = ea.shape[1]
    return pl.pallas_call(
        _edge_body,
        grid=(E // _EROWS,),
        in_specs=[
            pl.BlockSpec((_EROWS, HD), lambda i: (i, 0)),
            pl.BlockSpec((_EROWS, HD), lambda i: (i, 0)),
            pl.BlockSpec((_EROWS, ed), lambda i: (i, 0)),
            pl.BlockSpec((ed, HD), lambda i: (0, 0)),
            pl.BlockSpec((1, HD), lambda i: (0, 0)),
            pl.BlockSpec((HD, HD), lambda i: (0, 0)),
            pl.BlockSpec((1, HD), lambda i: (0, 0)),
        ],
        out_specs=pl.BlockSpec((_EROWS, HD), lambda i: (i, 0)),
        out_shape=jax.ShapeDtypeStruct((E, HD), jnp.float32),
    )(ga, gb, ea, wc, b1, w2, b2)


def _node_body(x_ref, ag_ref, w1a_ref, w1b_ref, b1_ref, w2_ref, b2_ref,
               out_ref):
    t = (
        jnp.dot(x_ref[...], w1a_ref[...], preferred_element_type=jnp.float32)
        + jnp.dot(ag_ref[...], w1b_ref[...], preferred_element_type=jnp.float32)
        + b1_ref[...]
    )
    t = jnp.maximum(t, 0.0)
    out_ref[...] = jnp.maximum(
        jnp.dot(t, w2_ref[...], preferred_element_type=jnp.float32) + b2_ref[...], 0.0
    )


def _node_tc(xin, aggr, w1a, w1b, b1, w2, b2):
    k = xin.shape[1]
    return pl.pallas_call(
        _node_body,
        grid=(N // _ROWS,),
        in_specs=[
            pl.BlockSpec((_ROWS, k), lambda i: (i, 0)),
            pl.BlockSpec((_ROWS, HD), lambda i: (i, 0)),
            pl.BlockSpec((k, HD), lambda i: (0, 0)),
            pl.BlockSpec((HD, HD), lambda i: (0, 0)),
            pl.BlockSpec((1, HD), lambda i: (0, 0)),
            pl.BlockSpec((HD, HD), lambda i: (0, 0)),
            pl.BlockSpec((1, HD), lambda i: (0, 0)),
        ],
        out_specs=pl.BlockSpec((_ROWS, HD), lambda i: (i, 0)),
        out_shape=jax.ShapeDtypeStruct((N, HD), jnp.float32),
    )(xin, aggr, w1a, w1b, b1, w2, b2)


def _pool_body(h_ref, batch_ref, g_ref, w1a_ref, w1b_ref, b1_ref, w2_ref, b2_ref,
               out_ref, sums_ref, counts_ref):
    i = pl.program_id(0)

    @pl.when(i == 0)
    def _init():
        sums_ref[...] = jnp.zeros_like(sums_ref)
        counts_ref[...] = jnp.zeros_like(counts_ref)

    b = batch_ref[...].reshape(1, _ROWS)
    seg = lax.broadcasted_iota(jnp.int32, (G, _ROWS), 0)
    onehot = (seg == b).astype(jnp.float32)
    sums_ref[...] += jnp.dot(onehot, h_ref[...], preferred_element_type=jnp.float32)
    cnt = jnp.sum(onehot, axis=1, keepdims=True)
    counts_ref[...] += jnp.broadcast_to(cnt, (G, 128))

    @pl.when(i == (N // _ROWS) - 1)
    def _final():
        c = counts_ref[...][:, 0:1]
        pooled = sums_ref[...] / jnp.maximum(c, 1.0)
        t = (
            jnp.dot(pooled, w1a_ref[...], preferred_element_type=jnp.float32)
            + jnp.dot(g_ref[...], w1b_ref[...], preferred_element_type=jnp.float32)
            + b1_ref[...]
        )
        t = jnp.maximum(t, 0.0)
        out_ref[...] = (
            jnp.dot(t, w2_ref[...], preferred_element_type=jnp.float32) + b2_ref[...]
        )


def _pool_tc(h, batch3d, gfeat, w1a, w1b, b1, w2, b2):
    gd = gfeat.shape[1]
    nc = w2.shape[1]
    return pl.pallas_call(
        _pool_body,
        grid=(N // _ROWS,),
        in_specs=[
            pl.BlockSpec((_ROWS, HD), lambda i: (i, 0)),
            pl.BlockSpec((1, 1, _ROWS), lambda i: (i, 0, 0)),
            pl.BlockSpec((G, gd), lambda i: (0, 0)),
            pl.BlockSpec((HD, HD), lambda i: (0, 0)),
            pl.BlockSpec((gd, HD), lambda i: (0, 0)),
            pl.BlockSpec((1, HD), lambda i: (0, 0)),
            pl.BlockSpec((HD, nc), lambda i: (0, 0)),
            pl.BlockSpec((1, nc), lambda i: (0, 0)),
        ],
        out_specs=pl.BlockSpec((G, nc), lambda i: (0, 0)),
        out_shape=jax.ShapeDtypeStruct((G, nc), jnp.float32),
        scratch_shapes=[
            pltpu.VMEM((G, HD), jnp.float32),
            pltpu.VMEM((G, 128), jnp.float32),
        ],
    )(h, batch3d, gfeat, w1a, w1b, b1, w2, b2)


_SC_MESH = plsc.VectorSubcoreMesh(core_axis_name="c", subcore_axis_name="s")
_NW = 32
_CH = 80

_G_PER_W = E // _NW
_G_CHUNKS = _G_PER_W // _CH

_S_PER_T = (E // 2) // 16
_S_CHUNKS = _S_PER_T // _CH
_ZROWS = 1000


@functools.partial(
    pl.kernel,
    mesh=_SC_MESH,
    out_type=[
        jax.ShapeDtypeStruct((E, HD), jnp.float32),
        jax.ShapeDtypeStruct((E, HD), jnp.float32),
    ],
    scratch_types=[
        pltpu.VMEM((_CH,), jnp.int32),
        pltpu.VMEM((_CH,), jnp.int32),
        pltpu.VMEM((_CH,), jnp.int32),
        pltpu.VMEM((_CH,), jnp.int32),
        pltpu.VMEM((_CH, HD), jnp.float32),
        pltpu.VMEM((_CH, HD), jnp.float32),
        pltpu.VMEM((_CH, HD), jnp.float32),
        pltpu.VMEM((_CH, HD), jnp.float32),
        pltpu.SemaphoreType.DMA((2,)),
        pltpu.SemaphoreType.DMA((2,)),
        pltpu.SemaphoreType.DMA((2,)),
        pltpu.SemaphoreType.DMA((2,)),
    ],
)
def _sc_gather(pa_hbm, pb_hbm, dst_hbm, src_hbm, ga_hbm, gb_hbm,
               idxd0, idxd1, idxs0, idxs1, ra0, ra1, rb0, rb1,
               sga, sgb, swa, swb):
    wid = lax.axis_index("s") * 2 + lax.axis_index("c")
    start = wid * _G_PER_W
    idxd = (idxd0, idxd1)
    idxs = (idxs0, idxs1)
    ra = (ra0, ra1)
    rb = (rb0, rb1)

    def load_idx_and_gather(c, p):
        base = start + c * _CH
        pltpu.sync_copy(dst_hbm.at[pl.ds(base, _CH)], idxd[p])
        pltpu.sync_copy(src_hbm.at[pl.ds(base, _CH)], idxs[p])
        pltpu.async_copy(pa_hbm.at[idxd[p]], ra[p], sga.at[p])
        pltpu.async_copy(pb_hbm.at[idxs[p]], rb[p], sgb.at[p])

    def wait_gather(p):
        pltpu.make_async_copy(ga_hbm.at[pl.ds(0, _CH)], ra[p], sga.at[p]).wait()
        pltpu.make_async_copy(gb_hbm.at[pl.ds(0, _CH)], rb[p], sgb.at[p]).wait()

    def issue_wb(c, p):
        base = start + c * _CH
        pltpu.async_copy(ra[p], ga_hbm.at[pl.ds(base, _CH)], swa.at[p])
        pltpu.async_copy(rb[p], gb_hbm.at[pl.ds(base, _CH)], swb.at[p])

    def wait_wb(p):
        pltpu.make_async_copy(ra[p], ga_hbm.at[pl.ds(0, _CH)], swa.at[p]).wait()
        pltpu.make_async_copy(rb[p], gb_hbm.at[pl.ds(0, _CH)], swb.at[p]).wait()

    load_idx_and_gather(0, 0)
    load_idx_and_gather(1, 1)

    def body(g, carry):
        c0 = 2 * g
        wait_gather(0)
        issue_wb(c0, 0)
        wait_gather(1)
        issue_wb(c0 + 1, 1)
        wait_wb(0)
        load_idx_and_gather(c0 + 2, 0)
        wait_wb(1)
        load_idx_and_gather(c0 + 3, 1)
        return carry

    lax.fori_loop(0, (_G_CHUNKS - 3) // 2, body, 0)

    wait_gather(0)
    issue_wb(_G_CHUNKS - 3, 0)
    wait_gather(1)
    issue_wb(_G_CHUNKS - 2, 1)
    wait_wb(0)
    load_idx_and_gather(_G_CHUNKS - 1, 0)
    wait_gather(0)
    issue_wb(_G_CHUNKS - 1, 0)
    wait_wb(0)
    wait_wb(1)


_OWN = 320
_NOUT = 32 * _OWN
_SCCH = 512
_SC_CHUNKS = E // _SCCH
_LROWS = _SCCH // 128
_REG = E + 128
_DUMP = E + 64
_FB = 80

_DN = jax.lax.GatherDimensionNumbers(
    offset_dims=(), collapsed_slice_dims=(0,), start_index_map=(0,)
)


_PB = 256
_PBLK = E // _PB


def _pos_body(dst_ref, pos_ref, cnt_ref, carry_ref):
    i = pl.program_id(0)

    @pl.when(i == 0)
    def _init():
        carry_ref[...] = jnp.zeros_like(carry_ref)

    d = dst_ref[...].reshape(1, _PB)
    owner = d // _OWN
    io32 = lax.broadcasted_iota(jnp.int32, (32, _PB), 0)
    onehot = (io32 == owner).astype(jnp.float32)
    r_io = lax.broadcasted_iota(jnp.int32, (_PB, _PB), 0)
    c_io = lax.broadcasted_iota(jnp.int32, (_PB, _PB), 1)
    lt = (r_io <= c_io).astype(jnp.float32)
    prefix = jnp.dot(onehot, lt, preferred_element_type=jnp.float32)
    carry = carry_ref[...][:, 0:1]
    pos_f = jnp.sum(onehot * (prefix + carry), axis=0, keepdims=True) - 1.0
    posg = owner * _REG + pos_f.astype(jnp.int32)
    pos_ref[...] = posg.reshape(1, 1, _PB)
    new_carry = carry + prefix[:, _PB - 1 : _PB]
    carry_ref[...] = jnp.broadcast_to(new_carry, (32, 128))

    @pl.when(i == _PBLK - 1)
    def _final():
        cnt_ref[...] = jnp.broadcast_to(new_carry, (32, 128))


def _pos_tc(dst3d):
    return pl.pallas_call(
        _pos_body,
        grid=(_PBLK,),
        in_specs=[pl.BlockSpec((1, 1, _PB), lambda i: (i, 0, 0))],
        out_specs=[
            pl.BlockSpec((1, 1, _PB), lambda i: (i, 0, 0)),
            pl.BlockSpec((32, 128), lambda i: (0, 0)),
        ],
        out_shape=[
            jax.ShapeDtypeStruct((_PBLK, 1, _PB), jnp.int32),
            jax.ShapeDtypeStruct((32, 128), jnp.float32),
        ],
        scratch_shapes=[pltpu.VMEM((32, 128), jnp.float32)],
    )(dst3d)


def _accumulate_list(m_hbm, dst_hbm, list_hbm, accf, idxb0, idxb1, dvb0, dvb1,
                     rowf0, rowf1, sgm, sgd, region0, node0, cnt):
    nfull = lax.div(cnt, _FB)
    rem = cnt - nfull * _FB
    cntv16 = jnp.full((16,), cnt, jnp.int32)
    iota = jax.lax.broadcasted_iota(jnp.int32, (16,), 0)
    idxb = (idxb0, idxb1)
    dvb = (dvb0, dvb1)
    rowf = (rowf0, rowf1)

    def add_row(p, g, l, r):
        base = r * HD
        for q in range(HD // 16):
            plsc.addupdate(
                accf.at[pl.ds(base + q * 16, 16)],
                rowf[p][g * 16 + l, pl.ds(q * 16, 16)],
            )

    def prefetch(ch, p):
        pltpu.sync_copy(list_hbm.at[pl.ds(region0 + ch * _FB, _FB)], idxb[p])
        pltpu.async_copy(m_hbm.at[idxb[p]], rowf[p], sgm.at[p])
        pltpu.async_copy(dst_hbm.at[idxb[p]], dvb[p], sgd.at[p])

    def process(p):
        pltpu.make_async_copy(m_hbm.at[pl.ds(0, _FB)], rowf[p], sgm.at[p]).wait()
        pltpu.make_async_copy(dst_hbm.at[pl.ds(0, _FB)], dvb[p], sgd.at[p]).wait()

        def acc_group(g, c2):
            dv = dvb[p][pl.ds(g * 16, 16)]
            for l in range(16):
                add_row(p, g, l, dv[l] - node0)
            return c2

        lax.fori_loop(0, _FB // 16, acc_group, 0)

    @pl.when(nfull > 0)
    def _p0():
        prefetch(0, 0)

    @pl.when(nfull > 1)
    def _p1():
        prefetch(1, 1)

    def pair(k, carry):
        b0 = 2 * k

        @pl.when(b0 < nfull)
        def _a():
            process(0)

        @pl.when(b0 + 2 < nfull)
        def _b():
            prefetch(b0 + 2, 0)

        @pl.when(b0 + 1 < nfull)
        def _c():
            process(1)

        @pl.when(b0 + 3 < nfull)
        def _d():
            prefetch(b0 + 3, 1)

        return carry

    lax.fori_loop(0, lax.div(nfull + 1, 2), pair, 0)

    @pl.when(rem > 0)
    def _tail():
        pltpu.sync_copy(list_hbm.at[pl.ds(region0 + nfull * _FB, _FB)], idxb0)
        for gq in range(_FB // 16):
            posv = nfull * _FB + gq * 16 + iota
            v = idxb0[pl.ds(gq * 16, 16)]
            idxb0[pl.ds(gq * 16, 16)] = jnp.where(posv < cntv16, v, 0)
        cpr = pltpu.async_copy(m_hbm.at[idxb0], rowf0, sgm.at[0])
        pltpu.sync_copy(dst_hbm.at[idxb0], dvb0)
        cpr.wait()

        def acc_group(g, c2):
            dv = dvb0[pl.ds(g * 16, 16)]
            for l in range(16):
                r = dv[l] - node0

                @pl.when(g * 16 + l < rem)
                def _(r=r, g=g, l=l):
                    add_row(0, g, l, r)
            return c2

        lax.fori_loop(0, _FB // 16, acc_group, 0)


def _zero_acc(accf):
    zero16f = jnp.zeros((16,), jnp.float32)

    def zacc(i, carry):
        accf[pl.ds(i * 16, 16)] = zero16f
        return carry

    lax.fori_loop(0, _OWN * HD // 16, zacc, 0)


_LW_CH = 80
_LW_PER_W = E // 32
_LW_CHUNKS = _LW_PER_W // _LW_CH


@functools.partial(
    pl.kernel,
    mesh=_SC_MESH,
    out_type=jax.ShapeDtypeStruct((32 * _REG,), jnp.int32),
    scratch_types=[
        pltpu.VMEM((_LW_CH,), jnp.int32),
        pltpu.VMEM((_LW_CH,), jnp.int32),
        pltpu.VMEM((_LW_CH,), jnp.int32),
        pltpu.VMEM((_LW_CH,), jnp.int32),
        pltpu.SemaphoreType.DMA((2,)),
    ],
)
def _sc_listwrite(pos_hbm, list_hbm, pb0, pb1, vb0, vb1, ssem):
    w = lax.axis_index("s") * 2 + lax.axis_index("c")
    start = w * _LW_PER_W
    pb = (pb0, pb1)
    vb = (vb0, vb1)
    iota = jax.lax.broadcasted_iota(jnp.int32, (16,), 0)

    def load_and_scatter(c, p):
        base = start + c * _LW_CH
        pltpu.sync_copy(pos_hbm.at[pl.ds(base, _LW_CH)], pb[p])
        for j in range(_LW_CH // 16):
            vb[p][pl.ds(j * 16, 16)] = base + j * 16 + iota
        pltpu.async_copy(vb[p], list_hbm.at[pb[p]], ssem.at[p])

    def wait_sc(p):
        pltpu.make_async_copy(vb[p], list_hbm.at[pl.ds(0, _LW_CH)],
                              ssem.at[p]).wait()

    load_and_scatter(0, 0)
    load_and_scatter(1, 1)

    def body(g, carry):
        c0 = 2 * g
        wait_sc(0)
        load_and_scatter(c0 + 2, 0)
        wait_sc(1)
        load_and_scatter(c0 + 3, 1)
        return carry

    lax.fori_loop(0, (_LW_CHUNKS - 3) // 2, body, 0)

    wait_sc(0)
    load_and_scatter(_LW_CHUNKS - 1, 0)
    wait_sc(1)
    wait_sc(0)


@functools.partial(
    pl.kernel,
    mesh=_SC_MESH,
    out_type=jax.ShapeDtypeStruct((_NOUT * HD,), jnp.float32),
    scratch_types=[
        pltpu.VMEM((_OWN * HD,), jnp.float32),
        pltpu.VMEM((16,), jnp.int32),
        pltpu.VMEM((_FB,), jnp.int32),
        pltpu.VMEM((_FB,), jnp.int32),
        pltpu.VMEM((_FB,), jnp.int32),
        pltpu.VMEM((_FB,), jnp.int32),
        pltpu.VMEM((_FB, HD), jnp.float32),
        pltpu.VMEM((_FB, HD), jnp.float32),
        pltpu.SemaphoreType.DMA((2,)),
        pltpu.SemaphoreType.DMA((2,)),
    ],
)
def _sc_scatter_replay(m_hbm, dst_hbm, list_hbm, cnt_hbm, aggr_hbm,
                       accf, cbuf, idxb0, idxb1, dvb0, dvb1, rowf0, rowf1,
                       sgm, sgd):
    w = lax.axis_index("s") * 2 + lax.axis_index("c")
    node0 = w * _OWN
    region0 = w * _REG

    _zero_acc(accf)
    pltpu.sync_copy(cnt_hbm.at[pl.ds(w * 16, 16)], cbuf)
    cnt = cbuf[pl.ds(0, 16)][0]
    _accumulate_list(m_hbm, dst_hbm, list_hbm, accf, idxb0, idxb1,
                     dvb0, dvb1, rowf0, rowf1, sgm, sgd,
                     region0, node0, cnt)
    pltpu.sync_copy(accf, aggr_hbm.at[pl.ds(node0 * HD, _OWN * HD)])


def _layer(xin, src, dst, edge_attr, lists, eW1, eb1, eW2, eb2, nW1, nb1, nW2, nb2):
    k = xin.shape[1]
    pa, pb = _pre_tc(xin, eW1[:k], eW1[k:2 * k])
    ga, gb = _sc_gather(pa, pb, dst, src)
    m = _edge_tc(ga, gb, edge_attr, eW1[2 * k:], eb1.reshape(1, HD), eW2,
                 eb2.reshape(1, HD))
    aggr_flat = _sc_scatter_replay(m, dst, lists[0], lists[1])
    aggr = aggr_flat.reshape(_NOUT, HD)[:N]
    return _node_tc(xin, aggr, nW1[:k], nW1[k:], nb1.reshape(1, HD), nW2,
                    nb2.reshape(1, HD))


def kernel(x, edge_index, edge_attr, batch, global_features,
           l1_eW1, l1_eb1, l1_eW2, l1_eb2, l1_nW1, l1_nb1, l1_nW2, l1_nb2,
           l2_eW1, l2_eb1, l2_eW2, l2_eb2, l2_nW1, l2_nb1, l2_nW2, l2_nb2,
           cW1, cb1, cW2, cb2):
    src = edge_index[0]
    dst = edge_index[1]

    pos3, cnts_f = _pos_tc(dst.reshape(_PBLK, 1, _PB))
    elist = _sc_listwrite(pos3.reshape(E))
    cnts = cnts_f[:, :16].astype(jnp.int32).reshape(32 * 16)
    lists = (elist, cnts)

    h = _layer(x, src, dst, edge_attr, lists,
               l1_eW1, l1_eb1, l1_eW2, l1_eb2, l1_nW1, l1_nb1,
               l1_nW2, l1_nb2)
    h = _layer(h, src, dst, edge_attr, lists,
               l2_eW1, l2_eb1, l2_eW2, l2_eb2, l2_nW1, l2_nb1,
               l2_nW2, l2_nb2)

    batch3d = batch.reshape(N // _ROWS, 1, _ROWS)
    nc = cW2.shape[1]
    return _pool_tc(h, batch3d, global_features, cW1[:HD], cW1[HD:],
                    cb1.reshape(1, HD), cW2, cb2.reshape(1, nc))
